# Initial kernel scaffold; baseline (speedup 1.0000x reference)
import jax
import jax.numpy as jnp
from jax.experimental import pallas as pl


def kernel(edge_attr, edge_attr2, triangle_1_1_1, triangle_1_1_2, triangle_1_2_2, triangle_2_2_2, inverse_edge_1, inverse_edge_2, mlps_W1, mlps_b1, mlps_W2, mlps_b2):
    raise NotImplementedError("write your pallas kernel here")


# speedup vs baseline: 2.1058x; 2.1058x over previous
import functools
import jax
import jax.numpy as jnp
from jax.experimental import pallas as pl

_E1 = 160000
_E2 = 320000
_IN = 128
_HID = 128
_BLK = 2000


def _mlp_multi_body(n_out, x_ref, *refs):
    x = x_ref[...]
    for k in range(n_out):
        w1, b1, w2, b2 = refs[4 * k:4 * k + 4]
        h = jnp.maximum(
            jnp.dot(x, w1[...], preferred_element_type=jnp.float32) + b1[...],
            0.0)
        o = jnp.dot(h, w2[...], preferred_element_type=jnp.float32) + b2[...]
        refs[4 * n_out + k][...] = o


def _mlp_tables(x, idxs, mlps_W1, mlps_b1, mlps_W2, mlps_b2):
    n = len(idxs)
    rows = x.shape[0]
    grid = (rows // _BLK,)
    row_spec = pl.BlockSpec((_BLK, _IN), lambda i: (i, 0))
    w_spec = pl.BlockSpec((_IN, _HID), lambda i: (0, 0))
    b_spec = pl.BlockSpec((1, _HID), lambda i: (0, 0))
    in_specs = [row_spec]
    args = [x]
    for k in idxs:
        args += [mlps_W1[k], mlps_b1[k].reshape(1, _HID),
                 mlps_W2[k], mlps_b2[k].reshape(1, _IN)]
        in_specs += [w_spec, b_spec, w_spec, b_spec]
    out_specs = [row_spec] * n
    out_shape = [jax.ShapeDtypeStruct((rows, _IN), jnp.float32)] * n
    fn = pl.pallas_call(
        functools.partial(_mlp_multi_body, n),
        grid=grid,
        in_specs=in_specs,
        out_specs=out_specs,
        out_shape=out_shape,
    )
    return fn(*args)


def kernel(edge_attr, edge_attr2, triangle_1_1_1, triangle_1_1_2,
           triangle_1_2_2, triangle_2_2_2, inverse_edge_1, inverse_edge_2,
           mlps_W1, mlps_b1, mlps_W2, mlps_b2):
    ij111, ik111, kj111 = triangle_1_1_1
    ij112, ik112, kj112 = triangle_1_1_2
    ij122, ik122, kj122 = triangle_1_2_2
    ij222, ik222, kj222 = triangle_2_2_2

    M0, M1 = _mlp_tables(edge_attr, [0, 1], mlps_W1, mlps_b1, mlps_W2, mlps_b2)
    M2, M3, M6, M7 = _mlp_tables(edge_attr2, [2, 3, 6, 7],
                                 mlps_W1, mlps_b1, mlps_W2, mlps_b2)

    ms111 = jax.ops.segment_sum(M0[ik111] * M0[kj111], ij111,
                                num_segments=_E1)
    ms112 = jax.ops.segment_sum(M1[ik112] * M2[kj112], ij112,
                                num_segments=_E1)
    ms122 = jax.ops.segment_sum(M3[ik122] * M3[kj122], ij122,
                                num_segments=_E1)
    out1 = edge_attr + ms111 + ms112 + ms112[inverse_edge_1] + ms122

    M4, M5 = _mlp_tables(out1, [4, 5], mlps_W1, mlps_b1, mlps_W2, mlps_b2)

    ms211 = jax.ops.segment_sum(M4[ij112] * M4[ik112], kj112,
                                num_segments=_E2)
    ms212 = jax.ops.segment_sum(M5[ij122] * M6[kj122], ik122,
                                num_segments=_E2)
    ms222 = jax.ops.segment_sum(M7[ik222] * M7[kj222], ij222,
                                num_segments=_E2)
    out2 = edge_attr2 + ms211 + ms212 + ms212[inverse_edge_2] + ms222
    return (out1, out2)
